# Initial kernel scaffold; baseline (speedup 1.0000x reference)
import functools

import jax
import jax.numpy as jnp
from jax import lax
from jax.experimental import pallas as pl
from jax.experimental.pallas import tpu as pltpu
from jax.experimental.pallas import tpu_sc as plsc

N = 10000
E = 320000
D = 128

NC = 2
NS = 16
NW = NC * NS
EPW = E // NW
CHUNK = 80
NCHUNKS = EPW // CHUNK
ROWS_PER_TILE = N // NS
OUT_CHUNK = 125

ROW_BLK = 1000
GRID = N // ROW_BLK


def _proj_body(x_ref, w_ref, b_ref, k_ref, q_ref, v_ref, s_ref):
    acc = jnp.dot(x_ref[...], w_ref[...], preferred_element_type=jnp.float32)
    acc = acc + b_ref[...]
    k_ref[...] = acc[:, 0 * D:1 * D]
    q_ref[...] = acc[:, 1 * D:2 * D]
    v_ref[...] = acc[:, 2 * D:3 * D]
    s_ref[...] = acc[:, 3 * D:4 * D]


def _edge_body(k_hbm, q_hbm, v_hbm, src_hbm, dst_hbm, agg_hbm,
               idx_s, idx_d, kd, qs, vs, stage, agg_sh, sem):
    c = lax.axis_index("c")
    s = lax.axis_index("s")
    wid = c * NS + s

    def _zrow(i, _):
        r = i // 8
        j = (i % 8) * 16
        stage[r, pl.ds(j, 16)] = jnp.zeros((16,), jnp.float32)
        return 0
    lax.fori_loop(0, OUT_CHUNK * 8, _zrow, 0)
    for g in range(ROWS_PER_TILE // OUT_CHUNK):
        pltpu.sync_copy(stage, agg_sh.at[pl.ds(s * ROWS_PER_TILE
                                               + g * OUT_CHUNK, OUT_CHUNK)])
    plsc.subcore_barrier()

    def _chunk(t, _):
        base = wid * EPW + t * CHUNK
        pltpu.sync_copy(src_hbm.at[pl.ds(base, CHUNK)], idx_s)
        pltpu.sync_copy(dst_hbm.at[pl.ds(base, CHUNK)], idx_d)
        cp_k = pltpu.async_copy(k_hbm.at[idx_d], kd, sem)
        cp_q = pltpu.async_copy(q_hbm.at[idx_s], qs, sem)
        cp_v = pltpu.async_copy(v_hbm.at[idx_s], vs, sem)
        cp_k.wait()
        cp_q.wait()
        cp_v.wait()

        def _row(i, _):
            r = i // 8
            j = (i % 8) * 16
            z = kd[r, pl.ds(j, 16)] + qs[r, pl.ds(j, 16)]
            eta = 1.0 / (1.0 + jnp.exp(-z))
            vs[r, pl.ds(j, 16)] = eta * vs[r, pl.ds(j, 16)]
            return 0
        lax.fori_loop(0, CHUNK * 8, _row, 0)

        pltpu.sync_copy(vs, agg_sh.at[idx_d], add=True)
        return 0
    lax.fori_loop(0, NCHUNKS, _chunk, 0)

    plsc.subcore_barrier()
    for g in range(ROWS_PER_TILE // OUT_CHUNK):
        off = s * ROWS_PER_TILE + g * OUT_CHUNK
        pltpu.sync_copy(agg_sh.at[pl.ds(off, OUT_CHUNK)], stage)
        pltpu.sync_copy(stage, agg_hbm.at[c, pl.ds(off, OUT_CHUNK)])


def _stats_body(agg_ref, skip_ref, h_ref, sum_ref, sq_ref):
    h = agg_ref[0] + agg_ref[1] + skip_ref[...]
    h_ref[...] = h

    @pl.when(pl.program_id(0) == 0)
    def _init():
        sum_ref[...] = jnp.zeros_like(sum_ref)
        sq_ref[...] = jnp.zeros_like(sq_ref)

    sum_ref[...] += jnp.sum(h, axis=0, keepdims=True)
    sq_ref[...] += jnp.sum(h * h, axis=0, keepdims=True)


def _norm_body(h_ref, sum_ref, sq_ref, gamma_ref, beta_ref, x_ref, o_ref):
    inv_n = 1.0 / N
    mean = sum_ref[...] * inv_n
    var = sq_ref[...] * inv_n - mean * mean
    scale = gamma_ref[...] * lax.rsqrt(var + 1e-5)
    h = (h_ref[...] - mean) * scale + beta_ref[...]
    o_ref[...] = jnp.maximum(h, 0.0) + x_ref[...]


def kernel(x, edge_index, Wk, bk, Wq, bq, Wv, bv, Wskip, bias, gamma, beta):
    w_all = jnp.concatenate(
        [Wk.T, Wq.T, Wv.T, Wskip.T], axis=1)
    b_all = jnp.concatenate(
        [bk, bq, bv, bias], axis=0).reshape(1, 4 * D)

    tab = jax.ShapeDtypeStruct((N, D), jnp.float32)
    k, q, v, skip = pl.pallas_call(
        _proj_body,
        grid=(GRID,),
        in_specs=[
            pl.BlockSpec((ROW_BLK, D), lambda i: (i, 0)),
            pl.BlockSpec((D, 4 * D), lambda i: (0, 0)),
            pl.BlockSpec((1, 4 * D), lambda i: (0, 0)),
        ],
        out_specs=[pl.BlockSpec((ROW_BLK, D), lambda i: (i, 0))] * 4,
        out_shape=[tab, tab, tab, tab],
    )(x, w_all, b_all)

    src = edge_index[0]
    dst = edge_index[1]

    mesh = plsc.VectorSubcoreMesh(
        core_axis_name="c", subcore_axis_name="s",
        num_cores=NC, num_subcores=NS)
    edge_fn = pl.kernel(
        _edge_body,
        out_type=jax.ShapeDtypeStruct((NC, N, D), jnp.float32),
        mesh=mesh,
        scratch_types=[
            pltpu.VMEM((CHUNK,), jnp.int32),
            pltpu.VMEM((CHUNK,), jnp.int32),
            pltpu.VMEM((CHUNK, D), jnp.float32),
            pltpu.VMEM((CHUNK, D), jnp.float32),
            pltpu.VMEM((CHUNK, D), jnp.float32),
            pltpu.VMEM((OUT_CHUNK, D), jnp.float32),
            pltpu.VMEM_SHARED((N, D), jnp.float32),
            pltpu.SemaphoreType.DMA,
        ],
    )
    agg = edge_fn(k, q, v, src, dst)

    h, hsum, hsq = pl.pallas_call(
        _stats_body,
        grid=(GRID,),
        in_specs=[
            pl.BlockSpec((NC, ROW_BLK, D), lambda i: (0, i, 0)),
            pl.BlockSpec((ROW_BLK, D), lambda i: (i, 0)),
        ],
        out_specs=[
            pl.BlockSpec((ROW_BLK, D), lambda i: (i, 0)),
            pl.BlockSpec((1, D), lambda i: (0, 0)),
            pl.BlockSpec((1, D), lambda i: (0, 0)),
        ],
        out_shape=[
            jax.ShapeDtypeStruct((N, D), jnp.float32),
            jax.ShapeDtypeStruct((1, D), jnp.float32),
            jax.ShapeDtypeStruct((1, D), jnp.float32),
        ],
    )(agg, skip)

    out = pl.pallas_call(
        _norm_body,
        grid=(GRID,),
        in_specs=[
            pl.BlockSpec((ROW_BLK, D), lambda i: (i, 0)),
            pl.BlockSpec((1, D), lambda i: (0, 0)),
            pl.BlockSpec((1, D), lambda i: (0, 0)),
            pl.BlockSpec((1, D), lambda i: (0, 0)),
            pl.BlockSpec((1, D), lambda i: (0, 0)),
            pl.BlockSpec((ROW_BLK, D), lambda i: (i, 0)),
        ],
        out_specs=pl.BlockSpec((ROW_BLK, D), lambda i: (i, 0)),
        out_shape=jax.ShapeDtypeStruct((N, D), jnp.float32),
    )(h, hsum, hsq, gamma.reshape(1, D), beta.reshape(1, D), x)

    return out


# speedup vs baseline: 3.7261x; 3.7261x over previous
import functools

import jax
import jax.numpy as jnp
from jax import lax
from jax.experimental import pallas as pl
from jax.experimental.pallas import tpu as pltpu
from jax.experimental.pallas import tpu_sc as plsc

N = 10000
E = 320000
D = 128

NC = 2
NS = 16
NW = NC * NS
EPW = E // NW
CHUNK = 80
NCHUNKS = EPW // CHUNK
OUT_CHUNK = 80
N_OUT_CHUNKS = N // OUT_CHUNK
OUT_ROUNDS = -(-N_OUT_CHUNKS // NS)

ROW_BLK = 1000
GRID = N // ROW_BLK


def _proj_body(x_ref, w_ref, b_ref, k_ref, q_ref, v_ref, s_ref):
    acc = jnp.dot(x_ref[...], w_ref[...], preferred_element_type=jnp.float32)
    acc = acc + b_ref[...]
    k_ref[...] = acc[:, 0 * D:1 * D]
    q_ref[...] = acc[:, 1 * D:2 * D]
    v_ref[...] = acc[:, 2 * D:3 * D]
    s_ref[...] = acc[:, 3 * D:4 * D]


def _edge_body(k_hbm, q_hbm, v_hbm, src_hbm, dst_hbm, agg_hbm,
               idx_s, idx_d, kd, qs, vs, stage, agg_sh, sem):
    c = lax.axis_index("c")
    s = lax.axis_index("s")
    wid = c * NS + s

    def _zrow(i, _):
        r = i // 8
        j = (i % 8) * 16
        stage[r, pl.ds(j, 16)] = jnp.zeros((16,), jnp.float32)
        return 0
    lax.fori_loop(0, OUT_CHUNK * 8, _zrow, 0)
    for g in range(OUT_ROUNDS):
        blk = g * NS + s

        @pl.when(blk < N_OUT_CHUNKS)
        def _zero_blk():
            pltpu.sync_copy(stage, agg_sh.at[pl.ds(blk * OUT_CHUNK,
                                                   OUT_CHUNK)])
    plsc.subcore_barrier()

    def _chunk(t, _):
        base = wid * EPW + t * CHUNK
        pltpu.sync_copy(src_hbm.at[pl.ds(base, CHUNK)], idx_s)
        pltpu.sync_copy(dst_hbm.at[pl.ds(base, CHUNK)], idx_d)
        cp_k = pltpu.async_copy(k_hbm.at[idx_d], kd, sem)
        cp_q = pltpu.async_copy(q_hbm.at[idx_s], qs, sem)
        cp_v = pltpu.async_copy(v_hbm.at[idx_s], vs, sem)
        cp_k.wait()
        cp_q.wait()
        cp_v.wait()

        def _row(i, _):
            r = i // 8
            j = (i % 8) * 16
            z = kd[r, pl.ds(j, 16)] + qs[r, pl.ds(j, 16)]
            eta = 1.0 / (1.0 + jnp.exp(-z))
            vs[r, pl.ds(j, 16)] = eta * vs[r, pl.ds(j, 16)]
            return 0
        lax.fori_loop(0, CHUNK * 8, _row, 0)

        pltpu.sync_copy(vs, agg_sh.at[idx_d], add=True)
        return 0
    lax.fori_loop(0, NCHUNKS, _chunk, 0)

    plsc.subcore_barrier()
    for g in range(OUT_ROUNDS):
        blk = g * NS + s

        @pl.when(blk < N_OUT_CHUNKS)
        def _copy_blk():
            off = blk * OUT_CHUNK
            pltpu.sync_copy(agg_sh.at[pl.ds(off, OUT_CHUNK)], stage)
            pltpu.sync_copy(stage, agg_hbm.at[c, pl.ds(off, OUT_CHUNK)])


def _stats_body(agg_ref, skip_ref, h_ref, sum_ref, sq_ref):
    h = agg_ref[0] + agg_ref[1] + skip_ref[...]
    h_ref[...] = h

    @pl.when(pl.program_id(0) == 0)
    def _init():
        sum_ref[...] = jnp.zeros_like(sum_ref)
        sq_ref[...] = jnp.zeros_like(sq_ref)

    sum_ref[...] += jnp.sum(h, axis=0, keepdims=True)
    sq_ref[...] += jnp.sum(h * h, axis=0, keepdims=True)


def _norm_body(h_ref, sum_ref, sq_ref, gamma_ref, beta_ref, x_ref, o_ref):
    inv_n = 1.0 / N
    mean = sum_ref[...] * inv_n
    var = sq_ref[...] * inv_n - mean * mean
    scale = gamma_ref[...] * lax.rsqrt(var + 1e-5)
    h = (h_ref[...] - mean) * scale + beta_ref[...]
    o_ref[...] = jnp.maximum(h, 0.0) + x_ref[...]


def kernel(x, edge_index, Wk, bk, Wq, bq, Wv, bv, Wskip, bias, gamma, beta):
    w_all = jnp.concatenate(
        [Wk.T, Wq.T, Wv.T, Wskip.T], axis=1)
    b_all = jnp.concatenate(
        [bk, bq, bv, bias], axis=0).reshape(1, 4 * D)

    tab = jax.ShapeDtypeStruct((N, D), jnp.float32)
    k, q, v, skip = pl.pallas_call(
        _proj_body,
        grid=(GRID,),
        in_specs=[
            pl.BlockSpec((ROW_BLK, D), lambda i: (i, 0)),
            pl.BlockSpec((D, 4 * D), lambda i: (0, 0)),
            pl.BlockSpec((1, 4 * D), lambda i: (0, 0)),
        ],
        out_specs=[pl.BlockSpec((ROW_BLK, D), lambda i: (i, 0))] * 4,
        out_shape=[tab, tab, tab, tab],
    )(x, w_all, b_all)

    src = edge_index[0]
    dst = edge_index[1]

    mesh = plsc.VectorSubcoreMesh(
        core_axis_name="c", subcore_axis_name="s",
        num_cores=NC, num_subcores=NS)
    edge_fn = pl.kernel(
        _edge_body,
        out_type=jax.ShapeDtypeStruct((NC, N, D), jnp.float32),
        mesh=mesh,
        scratch_types=[
            pltpu.VMEM((CHUNK,), jnp.int32),
            pltpu.VMEM((CHUNK,), jnp.int32),
            pltpu.VMEM((CHUNK, D), jnp.float32),
            pltpu.VMEM((CHUNK, D), jnp.float32),
            pltpu.VMEM((CHUNK, D), jnp.float32),
            pltpu.VMEM((OUT_CHUNK, D), jnp.float32),
            pltpu.VMEM_SHARED((N, D), jnp.float32),
            pltpu.SemaphoreType.DMA,
        ],
    )
    agg = edge_fn(k, q, v, src, dst)

    h, hsum, hsq = pl.pallas_call(
        _stats_body,
        grid=(GRID,),
        in_specs=[
            pl.BlockSpec((NC, ROW_BLK, D), lambda i: (0, i, 0)),
            pl.BlockSpec((ROW_BLK, D), lambda i: (i, 0)),
        ],
        out_specs=[
            pl.BlockSpec((ROW_BLK, D), lambda i: (i, 0)),
            pl.BlockSpec((1, D), lambda i: (0, 0)),
            pl.BlockSpec((1, D), lambda i: (0, 0)),
        ],
        out_shape=[
            jax.ShapeDtypeStruct((N, D), jnp.float32),
            jax.ShapeDtypeStruct((1, D), jnp.float32),
            jax.ShapeDtypeStruct((1, D), jnp.float32),
        ],
    )(agg, skip)

    out = pl.pallas_call(
        _norm_body,
        grid=(GRID,),
        in_specs=[
            pl.BlockSpec((ROW_BLK, D), lambda i: (i, 0)),
            pl.BlockSpec((1, D), lambda i: (0, 0)),
            pl.BlockSpec((1, D), lambda i: (0, 0)),
            pl.BlockSpec((1, D), lambda i: (0, 0)),
            pl.BlockSpec((1, D), lambda i: (0, 0)),
            pl.BlockSpec((ROW_BLK, D), lambda i: (i, 0)),
        ],
        out_specs=pl.BlockSpec((ROW_BLK, D), lambda i: (i, 0)),
        out_shape=jax.ShapeDtypeStruct((N, D), jnp.float32),
    )(h, hsum, hsq, gamma.reshape(1, D), beta.reshape(1, D), x)

    return out


# speedup vs baseline: 7.8304x; 2.1015x over previous
import functools

import jax
import jax.numpy as jnp
from jax import lax
from jax.experimental import pallas as pl
from jax.experimental.pallas import tpu as pltpu
from jax.experimental.pallas import tpu_sc as plsc

N = 10000
E = 320000
D = 128

NC = 2
NS = 16
NW = NC * NS
CHUNK = 64
NCHG = E // CHUNK
FULL_T = NCHG // NW
REM = NCHG - FULL_T * NW
PAIRS = FULL_T // 2
OUT_CHUNK = 40
N_OUT_CHUNKS = N // OUT_CHUNK
OUT_ROUNDS = -(-N_OUT_CHUNKS // NS)

ROW_BLK = 1000
GRID = N // ROW_BLK


def _proj_body(x_ref, w_ref, b_ref, k_ref, q_ref, v_ref, s_ref):
    acc = jnp.dot(x_ref[...], w_ref[...], preferred_element_type=jnp.float32)
    acc = acc + b_ref[...]
    k_ref[...] = acc[:, 0 * D:1 * D]
    q_ref[...] = acc[:, 1 * D:2 * D]
    v_ref[...] = acc[:, 2 * D:3 * D]
    s_ref[...] = acc[:, 3 * D:4 * D]


def _edge_body(k_hbm, q_hbm, v_hbm, src_hbm, dst_hbm, agg_hbm,
               idx_sa, idx_da, kda, qsa, vsa,
               idx_sb, idx_db, kdb, qsb, vsb,
               agg_sh, sema, semb):
    c = lax.axis_index("c")
    s = lax.axis_index("s")
    wid = c * NS + s
    stage = kda.at[pl.ds(0, OUT_CHUNK)]

    def _zrow(i, _):
        r = i // 8
        j = (i % 8) * 16
        stage[r, pl.ds(j, 16)] = jnp.zeros((16,), jnp.float32)
        return 0
    lax.fori_loop(0, OUT_CHUNK * 8, _zrow, 0)
    for g in range(OUT_ROUNDS):
        blk = g * NS + s

        @pl.when(blk < N_OUT_CHUNKS)
        def _zero_blk():
            pltpu.sync_copy(stage, agg_sh.at[pl.ds(blk * OUT_CHUNK,
                                                   OUT_CHUNK)])
    plsc.subcore_barrier()

    bufa = (idx_sa, idx_da, kda, qsa, vsa, sema)
    bufb = (idx_sb, idx_db, kdb, qsb, vsb, semb)

    def _fire(cid, buf):
        idx_s, idx_d, kd, qs, vs, sem = buf
        base = cid * CHUNK
        pltpu.sync_copy(src_hbm.at[pl.ds(base, CHUNK)], idx_s)
        pltpu.sync_copy(dst_hbm.at[pl.ds(base, CHUNK)], idx_d)
        pltpu.async_copy(k_hbm.at[idx_d], kd, sem)
        pltpu.async_copy(q_hbm.at[idx_s], qs, sem)
        pltpu.async_copy(v_hbm.at[idx_s], vs, sem)

    def _finish(buf):
        idx_s, idx_d, kd, qs, vs, sem = buf
        pltpu.make_async_copy(k_hbm.at[idx_d], kd, sem).wait()
        pltpu.make_async_copy(q_hbm.at[idx_s], qs, sem).wait()
        pltpu.make_async_copy(v_hbm.at[idx_s], vs, sem).wait()

        def _row(r, _):
            for j in range(8):
                sl = pl.ds(j * 16, 16)
                z = kd[r, sl] + qs[r, sl]
                eta = 1.0 / (1.0 + jnp.exp(-z))
                vs[r, sl] = eta * vs[r, sl]
            return 0
        lax.fori_loop(0, CHUNK, _row, 0)

        pltpu.sync_copy(vs, agg_sh.at[idx_d], add=True)

    _fire(wid, bufa)

    def _pair(p, _):
        _fire((2 * p + 1) * NW + wid, bufb)
        _finish(bufa)

        @pl.when(2 * p + 2 < FULL_T)
        def _refire():
            _fire((2 * p + 2) * NW + wid, bufa)
        _finish(bufb)
        return 0
    lax.fori_loop(0, PAIRS, _pair, 0)

    @pl.when(wid < REM)
    def _tail():
        _fire(FULL_T * NW + wid, bufa)
        _finish(bufa)

    plsc.subcore_barrier()
    for g in range(OUT_ROUNDS):
        blk = g * NS + s

        @pl.when(blk < N_OUT_CHUNKS)
        def _copy_blk():
            off = blk * OUT_CHUNK
            pltpu.sync_copy(agg_sh.at[pl.ds(off, OUT_CHUNK)], stage)
            pltpu.sync_copy(stage, agg_hbm.at[c, pl.ds(off, OUT_CHUNK)])


def _stats_body(agg_ref, skip_ref, h_ref, sum_ref, sq_ref):
    h = agg_ref[0] + agg_ref[1] + skip_ref[...]
    h_ref[...] = h

    @pl.when(pl.program_id(0) == 0)
    def _init():
        sum_ref[...] = jnp.zeros_like(sum_ref)
        sq_ref[...] = jnp.zeros_like(sq_ref)

    sum_ref[...] += jnp.sum(h, axis=0, keepdims=True)
    sq_ref[...] += jnp.sum(h * h, axis=0, keepdims=True)


def _norm_body(h_ref, sum_ref, sq_ref, gamma_ref, beta_ref, x_ref, o_ref):
    inv_n = 1.0 / N
    mean = sum_ref[...] * inv_n
    var = sq_ref[...] * inv_n - mean * mean
    scale = gamma_ref[...] * lax.rsqrt(var + 1e-5)
    h = (h_ref[...] - mean) * scale + beta_ref[...]
    o_ref[...] = jnp.maximum(h, 0.0) + x_ref[...]


def kernel(x, edge_index, Wk, bk, Wq, bq, Wv, bv, Wskip, bias, gamma, beta):
    w_all = jnp.concatenate(
        [Wk.T, Wq.T, Wv.T, Wskip.T], axis=1)
    b_all = jnp.concatenate(
        [bk, bq, bv, bias], axis=0).reshape(1, 4 * D)

    tab = jax.ShapeDtypeStruct((N, D), jnp.float32)
    k, q, v, skip = pl.pallas_call(
        _proj_body,
        grid=(GRID,),
        in_specs=[
            pl.BlockSpec((ROW_BLK, D), lambda i: (i, 0)),
            pl.BlockSpec((D, 4 * D), lambda i: (0, 0)),
            pl.BlockSpec((1, 4 * D), lambda i: (0, 0)),
        ],
        out_specs=[pl.BlockSpec((ROW_BLK, D), lambda i: (i, 0))] * 4,
        out_shape=[tab, tab, tab, tab],
    )(x, w_all, b_all)

    src = edge_index[0]
    dst = edge_index[1]

    mesh = plsc.VectorSubcoreMesh(
        core_axis_name="c", subcore_axis_name="s",
        num_cores=NC, num_subcores=NS)
    edge_fn = pl.kernel(
        _edge_body,
        out_type=jax.ShapeDtypeStruct((NC, N, D), jnp.float32),
        mesh=mesh,
        scratch_types=[
            pltpu.VMEM((CHUNK,), jnp.int32),
            pltpu.VMEM((CHUNK,), jnp.int32),
            pltpu.VMEM((CHUNK, D), jnp.float32),
            pltpu.VMEM((CHUNK, D), jnp.float32),
            pltpu.VMEM((CHUNK, D), jnp.float32),
            pltpu.VMEM((CHUNK,), jnp.int32),
            pltpu.VMEM((CHUNK,), jnp.int32),
            pltpu.VMEM((CHUNK, D), jnp.float32),
            pltpu.VMEM((CHUNK, D), jnp.float32),
            pltpu.VMEM((CHUNK, D), jnp.float32),
            pltpu.VMEM_SHARED((N, D), jnp.float32),
            pltpu.SemaphoreType.DMA,
            pltpu.SemaphoreType.DMA,
        ],
    )
    agg = edge_fn(k, q, v, src, dst)

    h, hsum, hsq = pl.pallas_call(
        _stats_body,
        grid=(GRID,),
        in_specs=[
            pl.BlockSpec((NC, ROW_BLK, D), lambda i: (0, i, 0)),
            pl.BlockSpec((ROW_BLK, D), lambda i: (i, 0)),
        ],
        out_specs=[
            pl.BlockSpec((ROW_BLK, D), lambda i: (i, 0)),
            pl.BlockSpec((1, D), lambda i: (0, 0)),
            pl.BlockSpec((1, D), lambda i: (0, 0)),
        ],
        out_shape=[
            jax.ShapeDtypeStruct((N, D), jnp.float32),
            jax.ShapeDtypeStruct((1, D), jnp.float32),
            jax.ShapeDtypeStruct((1, D), jnp.float32),
        ],
    )(agg, skip)

    out = pl.pallas_call(
        _norm_body,
        grid=(GRID,),
        in_specs=[
            pl.BlockSpec((ROW_BLK, D), lambda i: (i, 0)),
            pl.BlockSpec((1, D), lambda i: (0, 0)),
            pl.BlockSpec((1, D), lambda i: (0, 0)),
            pl.BlockSpec((1, D), lambda i: (0, 0)),
            pl.BlockSpec((1, D), lambda i: (0, 0)),
            pl.BlockSpec((ROW_BLK, D), lambda i: (i, 0)),
        ],
        out_specs=pl.BlockSpec((ROW_BLK, D), lambda i: (i, 0)),
        out_shape=jax.ShapeDtypeStruct((N, D), jnp.float32),
    )(h, hsum, hsq, gamma.reshape(1, D), beta.reshape(1, D), x)

    return out
